# initial kernel scaffold (unmeasured)
import jax
import jax.numpy as jnp
from jax import lax
from jax.experimental import pallas as pl
from jax.experimental.pallas import tpu as pltpu


def kernel(
    x,
):
    def body(*refs):
        pass

    out_shape = jax.ShapeDtypeStruct(..., jnp.float32)
    return pl.pallas_call(body, out_shape=out_shape)(...)


# baseline (device time: 14306 ns/iter reference)
import jax
import jax.numpy as jnp
from jax import lax
from jax.experimental import pallas as pl
from jax.experimental.pallas import tpu as pltpu

N_DEV = 32


def kernel(x):
    m, n = x.shape

    def body(x_ref, out_ref, comm_ref, send_sems, recv_sems):
        me = lax.axis_index("i")

        barrier_sem = pltpu.get_barrier_semaphore()
        for j in range(N_DEV):

            @pl.when(j != me)
            def _(j=j):
                pl.semaphore_signal(
                    barrier_sem,
                    inc=1,
                    device_id=(j,),
                    device_id_type=pl.DeviceIdType.MESH,
                )

        pl.semaphore_wait(barrier_sem, N_DEV - 1)

        xv = x_ref[...]

        v = xv
        while v.shape[0] > 1:
            h = v.shape[0] // 2
            v = v[:h] * v[h:]
        comm_ref[pl.ds(me, 1), :] = v

        sends = []
        for j in range(N_DEV):
            rdma = pltpu.make_async_remote_copy(
                src_ref=comm_ref.at[pl.ds(me, 1), :],
                dst_ref=comm_ref.at[pl.ds(me, 1), :],
                send_sem=send_sems.at[j],
                recv_sem=recv_sems.at[me],
                device_id=(j,),
                device_id_type=pl.DeviceIdType.MESH,
            )
            sends.append(rdma)

            @pl.when(j != me)
            def _(rdma=rdma):
                rdma.start()

        acc = xv
        shift = 1
        while shift < m:
            ones = jnp.ones((shift, n), jnp.float32)
            acc = acc * jnp.concatenate([ones, acc[: m - shift]], axis=0)
            shift *= 2

        for j in range(N_DEV):
            recv = pltpu.make_async_remote_copy(
                src_ref=comm_ref.at[pl.ds(j, 1), :],
                dst_ref=comm_ref.at[pl.ds(j, 1), :],
                send_sem=send_sems.at[j],
                recv_sem=recv_sems.at[j],
                device_id=(j,),
                device_id_type=pl.DeviceIdType.MESH,
            )

            @pl.when(j != me)
            def _(recv=recv, send=sends[j]):
                recv.wait_recv()
                send.wait_send()

        e = jnp.ones((1, n), jnp.float32)
        for j in range(N_DEV):
            pj = comm_ref[pl.ds(j, 1), :]
            e = e * jnp.where(j < me, pj, jnp.ones((1, n), jnp.float32))

        out_ref[...] = acc * e

    return pl.pallas_call(
        body,
        out_shape=jax.ShapeDtypeStruct((m, n), jnp.float32),
        in_specs=[pl.BlockSpec(memory_space=pltpu.VMEM)],
        out_specs=pl.BlockSpec(memory_space=pltpu.VMEM),
        scratch_shapes=[
            pltpu.VMEM((N_DEV, n), jnp.float32),
            pltpu.SemaphoreType.DMA((N_DEV,)),
            pltpu.SemaphoreType.DMA((N_DEV,)),
        ],
        compiler_params=pltpu.CompilerParams(collective_id=0),
    )(x)
